# baseline (speedup 1.0000x reference)
import functools

import jax
import jax.numpy as jnp
from jax import lax
from jax.experimental import pallas as pl

LAYERS = 6
HEADS = 16
DH = 64
PAIR = 128
CS = 512
CSA = 1024


def _ln_rows(v, g, b, eps=1e-5):
    m = jnp.mean(v, axis=-1, keepdims=True)
    ms = jnp.mean(v * v, axis=-1, keepdims=True)
    var = ms - m * m
    return (v - m) * lax.rsqrt(var + eps) * g + b


def _elu1(v):
    return jnp.where(v > 0, v + 1.0, jnp.exp(v))


def _pair_mask():
    r = lax.broadcasted_iota(jnp.int32, (PAIR, PAIR), 0) // DH
    c = lax.broadcasted_iota(jnp.int32, (PAIR, PAIR), 1) // DH
    return (r == c).astype(jnp.float32)


def _accum_kv(kf, v, kv_ref, ks_ref, npair):
    kfb = kf.astype(jnp.bfloat16)
    vb = v.astype(jnp.bfloat16)
    mask = _pair_mask()
    for p in range(npair):
        sl = slice(PAIR * p, PAIR * (p + 1))
        prod = lax.dot_general(kfb[:, sl], vb[:, sl],
                               (((0,), (0,)), ((), ())),
                               preferred_element_type=jnp.float32)
        kv_ref[0, sl, :] += prod * mask
    ks_ref[0, :, :] += jnp.sum(kf, axis=0, keepdims=True)


def _kv_pass_body(x_ref, wk_ref, bk_ref, wv_ref, bv_ref,
                  *refs, cpb, npair, ncast):
    cast_in = refs[:ncast]
    kv_ref, ks_ref = refs[ncast], refs[ncast + 1]
    cast_out = refs[ncast + 2:]
    i = pl.program_id(0)
    xb = x_ref[...].astype(jnp.bfloat16)
    k = jnp.dot(xb, wk_ref[...], preferred_element_type=jnp.float32) + bk_ref[...]
    v = jnp.dot(xb, wv_ref[...], preferred_element_type=jnp.float32) + bv_ref[...]
    kf = _elu1(k)

    @pl.when(i % cpb == 0)
    def _():
        kv_ref[...] = jnp.zeros_like(kv_ref)
        ks_ref[...] = jnp.zeros_like(ks_ref)

    _accum_kv(kf, v, kv_ref, ks_ref, npair)
    for src, dst in zip(cast_in, cast_out):
        dst[...] = src[...].astype(jnp.bfloat16)


def _layer_body(*refs, npair, last, cs, cpb, ncast):
    if last:
        (x_ref, wq_ref, bq_ref, kv_ref, ks_ref, wo_ref, bo_ref,
         w1_ref, b1_ref, w2_ref, b2_ref, g1_ref, be1_ref, g2_ref, be2_ref,
         bm_ref, bmt_ref, gf_ref, bf_ref, out_ref) = refs
        cast_in = cast_out = ()
    else:
        (x_ref, wq_ref, bq_ref, kv_ref, ks_ref, wo_ref, bo_ref,
         w1_ref, b1_ref, w2_ref, b2_ref, g1_ref, be1_ref, g2_ref, be2_ref,
         bm_ref, bmt_ref, wkn_ref, bkn_ref, wvn_ref, bvn_ref) = refs[:21]
        cast_in = refs[21:21 + ncast]
        out_ref, kvn_ref, ksn_ref = refs[21 + ncast:24 + ncast]
        cast_out = refs[24 + ncast:]
    i = pl.program_id(0)
    if not last:
        @pl.when(i % cpb == 0)
        def _():
            kvn_ref[...] = jnp.zeros_like(kvn_ref)
            ksn_ref[...] = jnp.zeros_like(ksn_ref)
    kvb = kv_ref[0].astype(jnp.bfloat16)
    ks = ks_ref[0]
    avs = []
    hb = cs // 4
    for half in range(4):
        rs = slice(half * hb, (half + 1) * hb)
        xb = x_ref[rs, :].astype(jnp.bfloat16)
        q = jnp.dot(xb, wq_ref[...], preferred_element_type=jnp.float32) + bq_ref[...]
        qf = _elu1(q)
        qfb = qf.astype(jnp.bfloat16)
        nums = []
        for p in range(npair):
            sl = slice(PAIR * p, PAIR * (p + 1))
            nums.append(jnp.dot(qfb[:, sl], kvb[sl, :],
                                preferred_element_type=jnp.float32))
        num = jnp.concatenate(nums, axis=1)
        den = jnp.dot((qf * ks).astype(jnp.bfloat16), bm_ref[...],
                      preferred_element_type=jnp.float32)
        z = 1.0 / (den + 1e-6)
        zf = jnp.dot(z.astype(jnp.bfloat16), bmt_ref[...],
                     preferred_element_type=jnp.float32)
        avs.append((num * zf).astype(jnp.bfloat16))
    av = jnp.concatenate(avs, axis=0)
    xv = x_ref[...]
    attn = jnp.dot(av, wo_ref[...],
                   preferred_element_type=jnp.float32) + bo_ref[...]
    x1 = xv + attn
    x1n = _ln_rows(x1, g1_ref[...], be1_ref[...])
    h = jnp.dot(x1n.astype(jnp.bfloat16), w1_ref[...],
                preferred_element_type=jnp.float32) + b1_ref[...]
    h = jnp.maximum(h, 0.0).astype(jnp.bfloat16)
    y = jnp.dot(h, w2_ref[...],
                preferred_element_type=jnp.float32) + b2_ref[...]
    x2 = _ln_rows(x1n + y, g2_ref[...], be2_ref[...])
    if last:
        x2 = _ln_rows(x2, gf_ref[...], bf_ref[...])
    else:
        x2b = x2.astype(jnp.bfloat16)
        kn = jnp.dot(x2b, wkn_ref[...],
                     preferred_element_type=jnp.float32) + bkn_ref[...]
        vn = jnp.dot(x2b, wvn_ref[...],
                     preferred_element_type=jnp.float32) + bvn_ref[...]
        _accum_kv(_elu1(kn), vn, kvn_ref, ksn_ref, npair)
    out_ref[...] = x2
    for src, dst in zip(cast_in, cast_out):
        dst[...] = src[...].astype(jnp.bfloat16)


def kernel(x, Wq, bq, Wk, bk, Wv, bv, Wo, bo, W1, b1, W2, b2, g1, be1, g2, be2, gF, bF):
    B, S, D = x.shape
    F = W1.shape[-1]
    cs = min(CS, S)
    csa = min(CSA, S)
    nb = B * S // cs
    cpb = S // cs
    nba = B * S // csa
    cpba = S // csa
    npair = D // PAIR
    x2 = x.reshape(B * S, D)

    di = jnp.arange(D, dtype=jnp.int32) // DH
    bm = (di[:, None] == jnp.arange(HEADS, dtype=jnp.int32)[None, :]).astype(jnp.bfloat16)
    bmt = bm.T

    row_spec = pl.BlockSpec((cs, D), lambda i: (i, 0))
    row_spec_a = pl.BlockSpec((csa, D), lambda i: (i, 0))
    full_mat = lambda shp: pl.BlockSpec(shp, lambda i: (0,) * len(shp))
    kv_spec = pl.BlockSpec((1, D, PAIR), lambda i: (i // cpb, 0, 0))
    ks_spec = pl.BlockSpec((1, 1, D), lambda i: (i // cpb, 0, 0))
    kv_spec_a = pl.BlockSpec((1, D, PAIR), lambda i: (i // cpba, 0, 0))
    ks_spec_a = pl.BlockSpec((1, 1, D), lambda i: (i // cpba, 0, 0))
    kv_shape = [jax.ShapeDtypeStruct((B, D, PAIR), jnp.float32),
                jax.ShapeDtypeStruct((B, 1, D), jnp.float32)]

    def cast_specs(shapes, n):
        ins, outs, outsh = [], [], []
        for (r, c) in shapes:
            ins.append(pl.BlockSpec((r // n, c), lambda i: (i, 0)))
            outs.append(pl.BlockSpec((r // n, c), lambda i: (i, 0)))
            outsh.append(jax.ShapeDtypeStruct((r, c), jnp.bfloat16))
        return ins, outs, outsh

    main_shapes = [(D, D), (D, D), (D, F), (F, D)]
    kvw_shapes = [(D, D), (D, D)]

    ci0, co0, csh0 = cast_specs(main_shapes + kvw_shapes, nba)
    kv_pass = pl.pallas_call(
        functools.partial(_kv_pass_body, cpb=cpba, npair=npair, ncast=6),
        grid=(nba,),
        in_specs=[row_spec_a, full_mat((D, D)), full_mat((1, D)),
                  full_mat((D, D)), full_mat((1, D))] + ci0,
        out_specs=[kv_spec_a, ks_spec_a] + co0,
        out_shape=kv_shape + csh0,
    )

    base_specs = [row_spec, full_mat((D, D)), full_mat((1, D)),
                  kv_spec, ks_spec,
                  full_mat((D, D)), full_mat((1, D)),
                  full_mat((D, F)), full_mat((1, F)),
                  full_mat((F, D)), full_mat((1, D)),
                  full_mat((1, D)), full_mat((1, D)),
                  full_mat((1, D)), full_mat((1, D)),
                  full_mat((D, HEADS)), full_mat((HEADS, D))]
    x_shape = jax.ShapeDtypeStruct((B * S, D), jnp.float32)

    def make_layer(last, ncast, cast_shapes):
        if last:
            in_specs = base_specs + [full_mat((1, D)), full_mat((1, D))]
            out_specs, out_shape = row_spec, x_shape
            body = functools.partial(_layer_body, npair=npair, last=True,
                                     cs=cs, cpb=cpb, ncast=0)
        else:
            ci, co, csh = cast_specs(cast_shapes, nb)
            in_specs = base_specs + [full_mat((D, D)), full_mat((1, D)),
                                     full_mat((D, D)), full_mat((1, D))] + ci
            out_specs = [row_spec, kv_spec, ks_spec] + co
            out_shape = [x_shape] + kv_shape + csh
            body = functools.partial(_layer_body, npair=npair, last=False,
                                     cs=cs, cpb=cpb, ncast=ncast)
        return pl.pallas_call(
            body, grid=(nb,), in_specs=in_specs,
            out_specs=out_specs, out_shape=out_shape,
        )

    wk0b = Wk[0].astype(jnp.bfloat16)
    wv0b = Wv[0].astype(jnp.bfloat16)
    kv, ksum, wqb, wob, w1b, w2b, wknb, wvnb = kv_pass(
        x2, wk0b, bk[0].reshape(1, D), wv0b, bv[0].reshape(1, D),
        Wq[0], Wo[0], W1[0], W2[0], Wk[1], Wv[1])
    for i in range(LAYERS):
        last = i == LAYERS - 1
        args = [x2, wqb, bq[i].reshape(1, D), kv, ksum,
                wob, bo[i].reshape(1, D),
                w1b, b1[i].reshape(1, F),
                w2b, b2[i].reshape(1, D),
                g1[i].reshape(1, D), be1[i].reshape(1, D),
                g2[i].reshape(1, D), be2[i].reshape(1, D),
                bm, bmt]
        if last:
            args += [gF.reshape(1, D), bF.reshape(1, D)]
            x2 = make_layer(True, 0, [])(*args)
        else:
            args += [wknb, bk[i + 1].reshape(1, D),
                     wvnb, bv[i + 1].reshape(1, D)]
            if i + 2 < LAYERS:
                srcs = [Wq[i + 1], Wo[i + 1], W1[i + 1], W2[i + 1],
                        Wk[i + 2], Wv[i + 2]]
                shapes = main_shapes + kvw_shapes
            else:
                srcs = [Wq[i + 1], Wo[i + 1], W1[i + 1], W2[i + 1]]
                shapes = main_shapes
            args += srcs
            res = make_layer(False, len(srcs), shapes)(*args)
            x2, kv, ksum = res[0], res[1], res[2]
            if i + 2 < LAYERS:
                wqb, wob, w1b, w2b, wknb, wvnb = res[3:]
            else:
                wqb, wob, w1b, w2b = res[3:]
                wknb = wvnb = None
    return x2.reshape(B, S, D)


# speedup vs baseline: 1.0421x; 1.0421x over previous
import functools

import jax
import jax.numpy as jnp
from jax import lax
from jax.experimental import pallas as pl

LAYERS = 6
HEADS = 16
DH = 64
PAIR = 128
CS = 512
CSA = 1024


def _ln_rows(v, g, b, eps=1e-5):
    m = jnp.mean(v, axis=-1, keepdims=True)
    ms = jnp.mean(v * v, axis=-1, keepdims=True)
    var = ms - m * m
    return (v - m) * lax.rsqrt(var + eps) * g + b


def _elu1(v):
    return jnp.where(v > 0, v + 1.0, jnp.exp(v))


def _pair_mask():
    r = lax.broadcasted_iota(jnp.int32, (PAIR, PAIR), 0) // DH
    c = lax.broadcasted_iota(jnp.int32, (PAIR, PAIR), 1) // DH
    return (r == c).astype(jnp.float32)


def _accum_kv(kf, v, kv_ref, ks_ref, npair):
    kfb = kf.astype(jnp.bfloat16)
    vb = v.astype(jnp.bfloat16)
    mask = _pair_mask()
    for p in range(npair):
        sl = slice(PAIR * p, PAIR * (p + 1))
        prod = lax.dot_general(kfb[:, sl], vb[:, sl],
                               (((0,), (0,)), ((), ())),
                               preferred_element_type=jnp.float32)
        kv_ref[0, sl, :] += prod * mask
    ks_ref[0, :, :] += jnp.sum(kf, axis=0, keepdims=True)


def _kv_pass_body(x_ref, wk_ref, bk_ref, wv_ref, bv_ref,
                  *refs, cpb, npair, ncast):
    cast_in = refs[:ncast]
    kv_ref, ks_ref = refs[ncast], refs[ncast + 1]
    cast_out = refs[ncast + 2:]
    i = pl.program_id(0)
    xb = x_ref[...].astype(jnp.bfloat16)
    k = jnp.dot(xb, wk_ref[...], preferred_element_type=jnp.float32) + bk_ref[...]
    v = jnp.dot(xb, wv_ref[...], preferred_element_type=jnp.float32) + bv_ref[...]
    kf = _elu1(k)

    @pl.when(i % cpb == 0)
    def _():
        kv_ref[...] = jnp.zeros_like(kv_ref)
        ks_ref[...] = jnp.zeros_like(ks_ref)

    _accum_kv(kf, v, kv_ref, ks_ref, npair)
    for src, dst in zip(cast_in, cast_out):
        dst[...] = src[...].astype(jnp.bfloat16)


def _layer_body(*refs, npair, last, cs, cpb, ncast):
    if last:
        (x_ref, wq_ref, bq_ref, kv_ref, ks_ref, wo_ref, bo_ref,
         w1_ref, b1_ref, w2_ref, b2_ref, g1_ref, be1_ref, g2_ref, be2_ref,
         bm_ref, bmt_ref, gf_ref, bf_ref, out_ref) = refs
        cast_in = cast_out = ()
    else:
        (x_ref, wq_ref, bq_ref, kv_ref, ks_ref, wo_ref, bo_ref,
         w1_ref, b1_ref, w2_ref, b2_ref, g1_ref, be1_ref, g2_ref, be2_ref,
         bm_ref, bmt_ref, wkn_ref, bkn_ref, wvn_ref, bvn_ref) = refs[:21]
        cast_in = refs[21:21 + ncast]
        out_ref, kvn_ref, ksn_ref = refs[21 + ncast:24 + ncast]
        cast_out = refs[24 + ncast:]
    i = pl.program_id(0)
    if not last:
        @pl.when(i % cpb == 0)
        def _():
            kvn_ref[...] = jnp.zeros_like(kvn_ref)
            ksn_ref[...] = jnp.zeros_like(ksn_ref)
    kvb = kv_ref[0].astype(jnp.bfloat16)
    ks = ks_ref[0]
    avs = []
    hb = cs // 2
    for half in range(2):
        rs = slice(half * hb, (half + 1) * hb)
        xb = x_ref[rs, :].astype(jnp.bfloat16)
        q = jnp.dot(xb, wq_ref[...], preferred_element_type=jnp.float32) + bq_ref[...]
        qf = _elu1(q)
        qfb = qf.astype(jnp.bfloat16)
        nums = []
        for p in range(npair):
            sl = slice(PAIR * p, PAIR * (p + 1))
            nums.append(jnp.dot(qfb[:, sl], kvb[sl, :],
                                preferred_element_type=jnp.float32))
        num = jnp.concatenate(nums, axis=1)
        den = jnp.dot((qf * ks).astype(jnp.bfloat16), bm_ref[...],
                      preferred_element_type=jnp.float32)
        z = 1.0 / (den + 1e-6)
        zf = jnp.dot(z.astype(jnp.bfloat16), bmt_ref[...],
                     preferred_element_type=jnp.float32)
        avs.append((num * zf).astype(jnp.bfloat16))
    av = jnp.concatenate(avs, axis=0)
    xv = x_ref[...]
    attn = jnp.dot(av, wo_ref[...],
                   preferred_element_type=jnp.float32) + bo_ref[...]
    x1 = xv + attn
    x1n = _ln_rows(x1, g1_ref[...], be1_ref[...])
    h = jnp.dot(x1n.astype(jnp.bfloat16), w1_ref[...],
                preferred_element_type=jnp.float32) + b1_ref[...]
    h = jnp.maximum(h, 0.0).astype(jnp.bfloat16)
    y = jnp.dot(h, w2_ref[...],
                preferred_element_type=jnp.float32) + b2_ref[...]
    x2 = _ln_rows(x1n + y, g2_ref[...], be2_ref[...])
    if last:
        x2 = _ln_rows(x2, gf_ref[...], bf_ref[...])
    else:
        x2b = x2.astype(jnp.bfloat16)
        kn = jnp.dot(x2b, wkn_ref[...],
                     preferred_element_type=jnp.float32) + bkn_ref[...]
        vn = jnp.dot(x2b, wvn_ref[...],
                     preferred_element_type=jnp.float32) + bvn_ref[...]
        _accum_kv(_elu1(kn), vn, kvn_ref, ksn_ref, npair)
    out_ref[...] = x2
    for src, dst in zip(cast_in, cast_out):
        dst[...] = src[...].astype(jnp.bfloat16)


def kernel(x, Wq, bq, Wk, bk, Wv, bv, Wo, bo, W1, b1, W2, b2, g1, be1, g2, be2, gF, bF):
    B, S, D = x.shape
    F = W1.shape[-1]
    cs = min(CS, S)
    csa = min(CSA, S)
    nb = B * S // cs
    cpb = S // cs
    nba = B * S // csa
    cpba = S // csa
    npair = D // PAIR
    x2 = x.reshape(B * S, D)

    di = jnp.arange(D, dtype=jnp.int32) // DH
    bm = (di[:, None] == jnp.arange(HEADS, dtype=jnp.int32)[None, :]).astype(jnp.bfloat16)
    bmt = bm.T

    row_spec = pl.BlockSpec((cs, D), lambda i: (i, 0))
    row_spec_a = pl.BlockSpec((csa, D), lambda i: (i, 0))
    full_mat = lambda shp: pl.BlockSpec(shp, lambda i: (0,) * len(shp))
    kv_spec = pl.BlockSpec((1, D, PAIR), lambda i: (i // cpb, 0, 0))
    ks_spec = pl.BlockSpec((1, 1, D), lambda i: (i // cpb, 0, 0))
    kv_spec_a = pl.BlockSpec((1, D, PAIR), lambda i: (i // cpba, 0, 0))
    ks_spec_a = pl.BlockSpec((1, 1, D), lambda i: (i // cpba, 0, 0))
    kv_shape = [jax.ShapeDtypeStruct((B, D, PAIR), jnp.float32),
                jax.ShapeDtypeStruct((B, 1, D), jnp.float32)]

    def cast_specs(shapes, n):
        ins, outs, outsh = [], [], []
        for (r, c) in shapes:
            ins.append(pl.BlockSpec((r // n, c), lambda i: (i, 0)))
            outs.append(pl.BlockSpec((r // n, c), lambda i: (i, 0)))
            outsh.append(jax.ShapeDtypeStruct((r, c), jnp.bfloat16))
        return ins, outs, outsh

    main_shapes = [(D, D), (D, D), (D, F), (F, D)]
    kvw_shapes = [(D, D), (D, D)]

    ci0, co0, csh0 = cast_specs(main_shapes + kvw_shapes, nba)
    kv_pass = pl.pallas_call(
        functools.partial(_kv_pass_body, cpb=cpba, npair=npair, ncast=6),
        grid=(nba,),
        in_specs=[row_spec_a, full_mat((D, D)), full_mat((1, D)),
                  full_mat((D, D)), full_mat((1, D))] + ci0,
        out_specs=[kv_spec_a, ks_spec_a] + co0,
        out_shape=kv_shape + csh0,
    )

    base_specs = [row_spec, full_mat((D, D)), full_mat((1, D)),
                  kv_spec, ks_spec,
                  full_mat((D, D)), full_mat((1, D)),
                  full_mat((D, F)), full_mat((1, F)),
                  full_mat((F, D)), full_mat((1, D)),
                  full_mat((1, D)), full_mat((1, D)),
                  full_mat((1, D)), full_mat((1, D)),
                  full_mat((D, HEADS)), full_mat((HEADS, D))]
    x_shape = jax.ShapeDtypeStruct((B * S, D), jnp.float32)

    def make_layer(last, ncast, cast_shapes):
        if last:
            in_specs = base_specs + [full_mat((1, D)), full_mat((1, D))]
            out_specs, out_shape = row_spec, x_shape
            body = functools.partial(_layer_body, npair=npair, last=True,
                                     cs=cs, cpb=cpb, ncast=0)
        else:
            ci, co, csh = cast_specs(cast_shapes, nb)
            in_specs = base_specs + [full_mat((D, D)), full_mat((1, D)),
                                     full_mat((D, D)), full_mat((1, D))] + ci
            out_specs = [row_spec, kv_spec, ks_spec] + co
            out_shape = [x_shape] + kv_shape + csh
            body = functools.partial(_layer_body, npair=npair, last=False,
                                     cs=cs, cpb=cpb, ncast=ncast)
        return pl.pallas_call(
            body, grid=(nb,), in_specs=in_specs,
            out_specs=out_specs, out_shape=out_shape,
        )

    wk0b = Wk[0].astype(jnp.bfloat16)
    wv0b = Wv[0].astype(jnp.bfloat16)
    kv, ksum, wqb, wob, w1b, w2b, wknb, wvnb = kv_pass(
        x2, wk0b, bk[0].reshape(1, D), wv0b, bv[0].reshape(1, D),
        Wq[0], Wo[0], W1[0], W2[0], Wk[1], Wv[1])
    for i in range(LAYERS):
        last = i == LAYERS - 1
        args = [x2, wqb, bq[i].reshape(1, D), kv, ksum,
                wob, bo[i].reshape(1, D),
                w1b, b1[i].reshape(1, F),
                w2b, b2[i].reshape(1, D),
                g1[i].reshape(1, D), be1[i].reshape(1, D),
                g2[i].reshape(1, D), be2[i].reshape(1, D),
                bm, bmt]
        if last:
            args += [gF.reshape(1, D), bF.reshape(1, D)]
            x2 = make_layer(True, 0, [])(*args)
        else:
            args += [wknb, bk[i + 1].reshape(1, D),
                     wvnb, bv[i + 1].reshape(1, D)]
            if i + 2 < LAYERS:
                srcs = [Wq[i + 1], Wo[i + 1], W1[i + 1], W2[i + 1],
                        Wk[i + 2], Wv[i + 2]]
                shapes = main_shapes + kvw_shapes
            else:
                srcs = [Wq[i + 1], Wo[i + 1], W1[i + 1], W2[i + 1]]
                shapes = main_shapes
            args += srcs
            res = make_layer(False, len(srcs), shapes)(*args)
            x2, kv, ksum = res[0], res[1], res[2]
            if i + 2 < LAYERS:
                wqb, wob, w1b, w2b, wknb, wvnb = res[3:]
            else:
                wqb, wob, w1b, w2b = res[3:]
                wknb = wvnb = None
    return x2.reshape(B, S, D)
